# baseline (speedup 1.0000x reference)
import functools

import jax
import jax.numpy as jnp
from jax import lax
from jax.experimental import pallas as pl
from jax.experimental.pallas import tpu as pltpu
from jax.experimental.pallas import tpu_sc as plsc

B = 16384
L = 20
N = 20
D = 64
LANES = 16
NC = 2
NS = 16
NW = NC * NS

ROWS_PER_W = B // NW
CHUNK = 32
NCHUNK = ROWS_PER_W // CHUNK
IDXW = (CHUNK * L) // 128
PART = N * LANES

TC_BLOCK = 256


def _cbow_sc_body(cw_hbm, neg_hbm, ctx_hbm, cen_hbm, out_hbm,
                  idx_c, idx_n, ctx_rows, neg_rows, dot_buf, sem_c, sem_n):
    wid = lax.axis_index("s") * NC + lax.axis_index("c")

    def chunk_body(c, carry):
        row0 = wid * ROWS_PER_W + c * CHUNK
        i0 = row0 * L

        for j in range(IDXW):
            pltpu.sync_copy(cw_hbm.at[pl.ds(i0 + j * 128, 128)], idx_c.at[j])
            pltpu.sync_copy(neg_hbm.at[pl.ds(i0 + j * 128, 128)], idx_n.at[j])

        copies = []
        for j in range(IDXW):
            copies.append(pltpu.async_copy(
                ctx_hbm.at[idx_c.at[j]],
                ctx_rows.at[pl.ds(j * 128, 128)], sem_c))
            copies.append(pltpu.async_copy(
                cen_hbm.at[idx_n.at[j]],
                neg_rows.at[pl.ds(j * 128, 128)], sem_n))
        for cp in copies:
            cp.wait()

        def row_body(i, carry2):
            base = i * L
            v = []
            for q in range(D // LANES):
                sl = pl.ds(q * LANES, LANES)
                acc = ctx_rows[base, sl]
                for j in range(1, L):
                    acc = acc + ctx_rows[base + j, sl]
                v.append(acc)
            for n in range(N):
                acc = v[0] * neg_rows[base + n, pl.ds(0, LANES)]
                for q in range(1, D // LANES):
                    sl = pl.ds(q * LANES, LANES)
                    acc = acc + v[q] * neg_rows[base + n, sl]
                dot_buf[pl.ds(i * PART + n * LANES, LANES)] = acc
            return carry2

        lax.fori_loop(0, CHUNK, row_body, 0)
        pltpu.sync_copy(dot_buf, out_hbm.at[pl.ds(row0 * PART, CHUNK * PART)])
        return carry

    lax.fori_loop(0, NCHUNK, chunk_body, 0)


def _reduce_tc_body(part_ref, mask_ref, out_ref):
    j = lax.broadcasted_iota(jnp.int32, (PART, N), 0)
    n = lax.broadcasted_iota(jnp.int32, (PART, N), 1)
    fold = jnp.where(j // LANES == n, 1.0, 0.0).astype(jnp.float32)
    red = jnp.dot(part_ref[...], fold, preferred_element_type=jnp.float32)
    out_ref[...] = red / mask_ref[...]


def kernel(center_words, context_negatives, mask_c, context_table, center_table):
    cw = center_words.astype(jnp.int32).reshape(B * L)
    neg = context_negatives.astype(jnp.int32).reshape(B * N)

    mesh = plsc.VectorSubcoreMesh(core_axis_name="c", subcore_axis_name="s")
    sc_run = functools.partial(
        pl.kernel,
        mesh=mesh,
        out_type=jax.ShapeDtypeStruct((B * PART,), jnp.float32),
        scratch_types=[
            pltpu.VMEM((IDXW, 128), jnp.int32),
            pltpu.VMEM((IDXW, 128), jnp.int32),
            pltpu.VMEM((CHUNK * L, D), jnp.float32),
            pltpu.VMEM((CHUNK * N, D), jnp.float32),
            pltpu.VMEM((CHUNK * PART,), jnp.float32),
            pltpu.SemaphoreType.DMA,
            pltpu.SemaphoreType.DMA,
        ],
        compiler_params=pltpu.CompilerParams(use_tc_tiling_on_sc=False),
    )(_cbow_sc_body)
    part = sc_run(cw, neg, context_table, center_table)

    out = pl.pallas_call(
        _reduce_tc_body,
        grid=(B // TC_BLOCK,),
        in_specs=[
            pl.BlockSpec((TC_BLOCK, PART), lambda i: (i, 0)),
            pl.BlockSpec((TC_BLOCK, 1), lambda i: (i, 0)),
        ],
        out_specs=pl.BlockSpec((TC_BLOCK, N), lambda i: (i, 0)),
        out_shape=jax.ShapeDtypeStruct((B, N), jnp.float32),
    )(part.reshape(B, PART), mask_c.reshape(B, 1))
    return out.reshape(B, 1, N)
